# Initial kernel scaffold; baseline (speedup 1.0000x reference)
import jax
import jax.numpy as jnp
from jax.experimental import pallas as pl


def kernel(node_feat, edge_index, emb_table, W1, b1, W2, b2):
    raise NotImplementedError("write your pallas kernel here")


# speedup vs baseline: 4.7474x; 4.7474x over previous
import functools

import jax
import jax.numpy as jnp
from jax import lax
from jax.experimental import pallas as pl
from jax.experimental.pallas import tpu as pltpu
from jax.experimental.pallas import tpu_sc as plsc

N = 10000
E = 320000
H = 128
V = 1000

NC = 2
NS = 16
NW = NC * NS
NP = 10240
EPW = E // NW
ECH = 80
NECH = EPW // ECH
GSZ = 5
RPW = NP // NW
GCH = 64
NGCH = RPW // GCH
RPS = NP // NS

_mesh = plsc.VectorSubcoreMesh(
    core_axis_name="c", subcore_axis_name="s", num_cores=NC, num_subcores=NS)

f32 = jnp.float32
i32 = jnp.int32


@functools.partial(
    pl.kernel,
    out_type=jax.ShapeDtypeStruct((NP, H), f32),
    mesh=_mesh,
    scratch_types=[
        pltpu.VMEM((GCH,), i32),
        pltpu.VMEM((GCH, H), f32),
        pltpu.SemaphoreType.DMA,
    ],
)
def _sc_emb(feat_hbm, emb_hbm, h0_hbm, fidx_v, rows_v, sem):
  cid = lax.axis_index("c")
  sid = lax.axis_index("s")
  wid = sid * NC + cid

  def emb_body(j, carry):
    rb = wid * RPW + j * GCH
    pltpu.sync_copy(feat_hbm.at[pl.ds(rb, GCH)], fidx_v)
    pltpu.async_copy(emb_hbm.at[fidx_v], rows_v, sem).wait()
    pltpu.sync_copy(rows_v, h0_hbm.at[pl.ds(rb, GCH)])
    return carry

  lax.fori_loop(0, NGCH, emb_body, 0)


@functools.partial(
    pl.kernel,
    out_type=jax.ShapeDtypeStruct((NC * NP, H), f32),
    mesh=_mesh,
    scratch_types=[
        pltpu.VMEM((ECH,), i32),
        pltpu.VMEM((ECH,), i32),
        pltpu.VMEM((ECH, H), f32),
        pltpu.VMEM((ECH, H), f32),
        pltpu.VMEM_SHARED((NP, H), f32),
    ],
)
def _sc_deg(src_hbm, dst_hbm, ones_a_hbm, ones_b_hbm, zrows_hbm, out_hbm,
            sidx_v, didx_v, ones_a_v, ones_b_v, acc_sp):
  cid = lax.axis_index("c")
  sid = lax.axis_index("s")
  wid = sid * NC + cid

  @pl.when(sid == 0)
  def _():
    pltpu.sync_copy(zrows_hbm, acc_sp)

  pltpu.sync_copy(ones_a_hbm, ones_a_v)
  pltpu.sync_copy(ones_b_hbm, ones_b_v)
  plsc.subcore_barrier()

  def body(i, carry):
    base = wid * EPW + i * ECH
    pltpu.sync_copy(src_hbm.at[pl.ds(base, ECH)], sidx_v)
    pltpu.sync_copy(ones_a_v, acc_sp.at[sidx_v], add=True)
    pltpu.sync_copy(dst_hbm.at[pl.ds(base, ECH)], didx_v)
    pltpu.sync_copy(ones_b_v, acc_sp.at[didx_v], add=True)
    return carry

  lax.fori_loop(0, NECH, body, 0)

  plsc.subcore_barrier()
  out_base = cid * NP + sid * RPS
  pltpu.sync_copy(acc_sp.at[pl.ds(sid * RPS, RPS)],
                  out_hbm.at[pl.ds(out_base, RPS)])


@functools.partial(
    pl.kernel,
    out_type=jax.ShapeDtypeStruct((NC * NP, H), f32),
    mesh=_mesh,
    scratch_types=[
        pltpu.VMEM((ECH,), i32),
        pltpu.VMEM((ECH,), i32),
        pltpu.VMEM((ECH,), i32),
        pltpu.VMEM((ECH,), i32),
        pltpu.VMEM((ECH, H), f32),
        pltpu.VMEM((ECH, H), f32),
        pltpu.SemaphoreType.DMA,
        pltpu.SemaphoreType.DMA,
        pltpu.VMEM_SHARED((NP, H), f32),
    ],
)
def _sc_segsum(g_hbm, src_hbm, dst_hbm, zrows_hbm, out_hbm,
               sa_v, da_v, sb_v, db_v, rows_a, rows_b, sem_a, sem_b, m_sp):
  cid = lax.axis_index("c")
  sid = lax.axis_index("s")
  wid = sid * NC + cid

  pltpu.sync_copy(zrows_hbm.at[pl.ds(sid * RPS, RPS)],
                  m_sp.at[pl.ds(sid * RPS, RPS)])
  base = wid * EPW
  plsc.subcore_barrier()

  sbufs = (sa_v, sb_v)
  dbufs = (da_v, db_v)
  rbufs = (rows_a, rows_b)
  sems = (sem_a, sem_b)

  def group(g, carry):
    c0 = base + g * GSZ * ECH
    pltpu.sync_copy(src_hbm.at[pl.ds(c0, ECH)], sa_v)
    pltpu.sync_copy(dst_hbm.at[pl.ds(c0, ECH)], da_v)
    pend = pltpu.async_copy(g_hbm.at[sa_v], rows_a, sem_a)
    for k in range(1, GSZ):
      ck = base + (g * GSZ + k) * ECH
      pltpu.sync_copy(src_hbm.at[pl.ds(ck, ECH)], sbufs[k % 2])
      pltpu.sync_copy(dst_hbm.at[pl.ds(ck, ECH)], dbufs[k % 2])
      nxt = pltpu.async_copy(g_hbm.at[sbufs[k % 2]], rbufs[k % 2], sems[k % 2])
      pend.wait()
      pltpu.sync_copy(rbufs[(k - 1) % 2], m_sp.at[dbufs[(k - 1) % 2]],
                      add=True)
      pend = nxt
    pend.wait()
    pltpu.sync_copy(rbufs[(GSZ - 1) % 2], m_sp.at[dbufs[(GSZ - 1) % 2]],
                    add=True)
    return carry

  lax.fori_loop(0, NECH // GSZ, group, 0)

  plsc.subcore_barrier()
  out_base = cid * NP + sid * RPS
  pltpu.sync_copy(m_sp.at[pl.ds(sid * RPS, RPS)],
                  out_hbm.at[pl.ds(out_base, RPS)])


RB = 512
NB = NP // RB


def _scale_body(h_ref, ns_ref, o_ref):
  o_ref[...] = h_ref[...] * ns_ref[...]


_scale = pl.pallas_call(
    _scale_body,
    grid=(NB,),
    in_specs=[
        pl.BlockSpec((RB, H), lambda i: (i, 0)),
        pl.BlockSpec((RB, 1), lambda i: (i, 0)),
    ],
    out_specs=pl.BlockSpec((RB, H), lambda i: (i, 0)),
    out_shape=jax.ShapeDtypeStruct((NP, H), f32),
)


def _layer1_body(mp_ref, nd_ref, ns_ref, w_ref, b_ref, o_ref):
  m = (mp_ref[0] + mp_ref[1]) * nd_ref[...]
  y = jnp.dot(m, w_ref[...], preferred_element_type=f32) + b_ref[...]
  o_ref[...] = jnp.maximum(y, 0.0) * ns_ref[...]


_layer1 = pl.pallas_call(
    _layer1_body,
    grid=(NB,),
    in_specs=[
        pl.BlockSpec((NC, RB, H), lambda i: (0, i, 0)),
        pl.BlockSpec((RB, 1), lambda i: (i, 0)),
        pl.BlockSpec((RB, 1), lambda i: (i, 0)),
        pl.BlockSpec((H, H), lambda i: (0, 0)),
        pl.BlockSpec((1, H), lambda i: (0, 0)),
    ],
    out_specs=pl.BlockSpec((RB, H), lambda i: (i, 0)),
    out_shape=jax.ShapeDtypeStruct((NP, H), f32),
)


def _layer2_body(mp_ref, nd_ref, w_ref, b_ref, o_ref):
  i = pl.program_id(0)
  m = (mp_ref[0] + mp_ref[1]) * nd_ref[...]
  y = jnp.dot(m, w_ref[...], preferred_element_type=f32) + b_ref[...]
  y = jnp.maximum(y, 0.0)
  rows = lax.broadcasted_iota(i32, (RB, 1), 0) + i * RB
  y = jnp.where(rows < N, y, 0.0)
  part = jnp.sum(y, axis=0, keepdims=True)

  @pl.when(i == 0)
  def _():
    o_ref[...] = jnp.zeros_like(o_ref)

  o_ref[...] += part

  @pl.when(i == NB - 1)
  def _():
    o_ref[...] *= (1.0 / N)


_layer2 = pl.pallas_call(
    _layer2_body,
    grid=(NB,),
    in_specs=[
        pl.BlockSpec((NC, RB, H), lambda i: (0, i, 0)),
        pl.BlockSpec((RB, 1), lambda i: (i, 0)),
        pl.BlockSpec((H, H), lambda i: (0, 0)),
        pl.BlockSpec((1, H), lambda i: (0, 0)),
    ],
    out_specs=pl.BlockSpec((1, H), lambda i: (0, 0)),
    out_shape=jax.ShapeDtypeStruct((1, H), f32),
)


@jax.jit
def kernel(node_feat, edge_index, emb_table, W1, b1, W2, b2):
  src = edge_index[0]
  dst = edge_index[1]
  feat_p = jnp.concatenate(
      [node_feat, jnp.zeros((NP - N,), dtype=i32)])
  zrows = jnp.zeros((NP, H), dtype=f32)
  col = jnp.arange(H)[None, :]
  ones_a = jnp.broadcast_to((col < 64).astype(f32), (ECH, H))
  ones_b = jnp.broadcast_to((col >= 64).astype(f32), (ECH, H))

  h0 = _sc_emb(feat_p, emb_table)
  dg_p = _sc_deg(src, dst, ones_a, ones_b, zrows)

  deg_out = dg_p[:NP, 0] + dg_p[NP:, 0]
  deg_in = dg_p[:NP, 64] + dg_p[NP:, 64]
  ns = jnp.where(deg_out > 0,
                 lax.rsqrt(jnp.maximum(deg_out, 1e-12)), 0.0)[:, None]
  nd = jnp.where(deg_in > 0,
                 lax.rsqrt(jnp.maximum(deg_in, 1e-12)), 0.0)[:, None]

  g1 = _scale(h0, ns)
  m1p = _sc_segsum(g1, src, dst, zrows).reshape(NC, NP, H)
  g2 = _layer1(m1p, nd, ns, W1, b1.reshape(1, H))
  m2p = _sc_segsum(g2, src, dst, zrows).reshape(NC, NP, H)
  return _layer2(m2p, nd, W2, b2.reshape(1, H))
